# baseline (speedup 1.0000x reference)
import functools

import jax
import jax.numpy as jnp
from jax import lax
from jax.experimental import pallas as pl
from jax.experimental.pallas import tpu as pltpu
from jax.experimental.pallas import tpu_sc as plsc

N = 10000
E = 160000
DN = 128
DE = 16
HID = 128
NUM_MP = 6

NPAD = 10240
BI = 256
EPAD = 163840
ROW_BLK = 2000
EROW_BLK = 2048


def _mm_bias_body(x_ref, w_ref, b_ref, o_ref):
    o_ref[...] = lax.dot_general(
        x_ref[...], w_ref[...], (((1,), (1,)), ((), ()))) + b_ref[...]


def _mm_bias(x, w, b, row_blk):
    n, _ = x.shape
    dout = w.shape[0]
    grid = n // row_blk
    return pl.pallas_call(
        _mm_bias_body,
        grid=(grid,),
        in_specs=[
            pl.BlockSpec((row_blk, x.shape[1]), lambda i: (i, 0)),
            pl.BlockSpec(w.shape, lambda i: (0, 0)),
            pl.BlockSpec((1, dout), lambda i: (0, 0)),
        ],
        out_specs=pl.BlockSpec((row_blk, dout), lambda i: (i, 0)),
        out_shape=jax.ShapeDtypeStruct((n, dout), jnp.float32),
    )(x, w, b.reshape(1, dout))


def _combine_mm_body(sbp_ref, t0_ref, t1_ref, u_ref, w_ref, s_ref, up_ref):
    s = sbp_ref[...] + t0_ref[...] + t1_ref[...] - u_ref[...]
    s_ref[...] = s
    up_ref[...] = lax.dot_general(s, w_ref[...], (((1,), (1,)), ((), ())))


def _combine_mm(sbp, t0, t1, u, wh):
    grid = N // ROW_BLK
    blk = pl.BlockSpec((ROW_BLK, HID), lambda i: (i, 0))
    return pl.pallas_call(
        _combine_mm_body,
        grid=(grid,),
        in_specs=[blk, blk, blk, blk,
                  pl.BlockSpec((HID, HID), lambda i: (0, 0))],
        out_specs=[blk, blk],
        out_shape=[jax.ShapeDtypeStruct((N, HID), jnp.float32),
                   jax.ShapeDtypeStruct((N, HID), jnp.float32)],
    )(sbp, t0, t1, u, wh)


def _sum4_body(a_ref, b_ref, c_ref, d_ref, o_ref):
    o_ref[...] = a_ref[...] + b_ref[...] + c_ref[...] + d_ref[...]


def _sum4(a, b, c, d):
    grid = N // ROW_BLK
    blk = pl.BlockSpec((ROW_BLK, HID), lambda i: (i, 0))
    return pl.pallas_call(
        _sum4_body,
        grid=(grid,),
        in_specs=[blk, blk, blk, blk],
        out_specs=blk,
        out_shape=jax.ShapeDtypeStruct((N, HID), jnp.float32),
    )(a, b, c, d)


def _mm2_body(a_ref, b_ref, w_ref, o_ref):
    o_ref[...] = lax.dot_general(
        a_ref[...] + b_ref[...], w_ref[...], (((1,), (1,)), ((), ())))


def _mm2(a, b, w):
    grid = N // ROW_BLK
    blk = pl.BlockSpec((ROW_BLK, HID), lambda i: (i, 0))
    return pl.pallas_call(
        _mm2_body,
        grid=(grid,),
        in_specs=[blk, blk, pl.BlockSpec((HID, HID), lambda i: (0, 0))],
        out_specs=blk,
        out_shape=jax.ShapeDtypeStruct((N, HID), jnp.float32),
    )(a, b, w)


def _head_body(p0_ref, s6_ref, w_ref, b_ref, h_ref):
    pre = p0_ref[...] + s6_ref[...]
    h = lax.dot_general(pre, w_ref[...], (((1,), (1,)), ((), ()))) + b_ref[...]
    h_ref[...] = jnp.maximum(h, 0.0)


def _head(p0, s6, wo, wob):
    grid = N // ROW_BLK
    blk = pl.BlockSpec((ROW_BLK, HID), lambda i: (i, 0))
    return pl.pallas_call(
        _head_body,
        grid=(grid,),
        in_specs=[blk, blk, pl.BlockSpec((HID, HID), lambda i: (0, 0)),
                  pl.BlockSpec((1, HID), lambda i: (0, 0))],
        out_specs=blk,
        out_shape=jax.ShapeDtypeStruct((N, HID), jnp.float32),
    )(p0, s6, wo, wob.reshape(1, HID))


def _attn_stats_body(hb_ref, hf_ref, stats_ref):
    hb = hb_ref[...]
    hf = hf_ref[...]
    s = lax.dot_general(hb, hf, (((1,), (1,)), ((), ())))
    col = lax.broadcasted_iota(jnp.int32, s.shape, 1)
    s = jnp.where(col < N, s, -jnp.inf)
    m = jnp.max(s, axis=1)
    z = jnp.sum(jnp.exp(s - m[:, None]), axis=1)
    stats_ref[0:8, :] = jnp.broadcast_to(m[None, :], (8, BI))
    stats_ref[8:16, :] = jnp.broadcast_to(z[None, :], (8, BI))


def _attn_stats(hpad):
    grid = NPAD // BI
    return pl.pallas_call(
        _attn_stats_body,
        grid=(grid,),
        in_specs=[pl.BlockSpec((BI, HID), lambda i: (i, 0)),
                  pl.BlockSpec((NPAD, HID), lambda i: (0, 0))],
        out_specs=pl.BlockSpec((16, BI), lambda i: (0, i)),
        out_shape=jax.ShapeDtypeStruct((16, NPAD), jnp.float32),
    )(hpad, hpad)


def _attn_out_body(hb_ref, hf_ref, stats_ref, o_ref):
    hb = hb_ref[...]
    hf = hf_ref[...]
    s = lax.dot_general(hb, hf, (((1,), (1,)), ((), ())))
    m = stats_ref[0:1, :]
    rz = stats_ref[8:9, :]
    p = jnp.exp(s - m) * rz
    o_ref[...] = lax.dot_general(p, hf, (((1,), (0,)), ((), ()))) + hb


def _attn_out(hpad, stats):
    grid = NPAD // BI
    return pl.pallas_call(
        _attn_out_body,
        grid=(grid,),
        in_specs=[pl.BlockSpec((BI, HID), lambda i: (i, 0)),
                  pl.BlockSpec((NPAD, HID), lambda i: (0, 0)),
                  pl.BlockSpec((16, NPAD), lambda i: (0, 0))],
        out_specs=pl.BlockSpec((BI, HID), lambda i: (i, 0)),
        out_shape=jax.ShapeDtypeStruct((NPAD, HID), jnp.float32),
    )(hpad, hpad, stats)


NACC = 10112
TROWS = NACC // 16
NCHUNK = 40
EC = 80


def _s0_body(np_hbm, ep_hbm, srcidx, dstidx, z_hbm,
             out_hbm, idxs_v, idxd_v, grows_v, erows_v, acc, sem):
    c = lax.axis_index("c")
    s = lax.axis_index("s")
    wid = s * 2 + c
    r0 = s * TROWS
    pltpu.sync_copy(z_hbm.at[pl.ds(r0, TROWS)], acc.at[pl.ds(r0, TROWS)])
    pltpu.sync_copy(srcidx.at[wid], idxs_v)
    pltpu.sync_copy(dstidx.at[wid], idxd_v)
    plsc.subcore_barrier()

    def chunk(j, carry):
        gd = pltpu.async_copy(np_hbm.at[idxs_v.at[j]], grows_v, sem)
        pltpu.sync_copy(
            ep_hbm.at[pl.ds(wid * (NCHUNK * 128) + j * 128, 128)], erows_v)
        gd.wait()

        def ew(r, carry2):
            for c8 in range(8):
                sl = pl.ds(c8 * 16, 16)
                grows_v[r, sl] = jnp.maximum(grows_v[r, sl] + erows_v[r, sl], 0.0)
            return carry2

        lax.fori_loop(0, 128, ew, 0)
        pltpu.sync_copy(grows_v, acc.at[idxd_v.at[j]], add=True)
        return carry

    lax.fori_loop(0, NCHUNK, chunk, 0)
    plsc.subcore_barrier()
    pltpu.sync_copy(acc.at[pl.ds(r0, TROWS)], out_hbm.at[c, pl.ds(r0, TROWS)])


_s0_call = pl.kernel(
    _s0_body,
    out_type=jax.ShapeDtypeStruct((2, NACC, HID), jnp.float32),
    mesh=plsc.VectorSubcoreMesh(core_axis_name="c", subcore_axis_name="s"),
    scratch_types=[
        pltpu.VMEM((NCHUNK, 128), jnp.int32),
        pltpu.VMEM((NCHUNK, 128), jnp.int32),
        pltpu.VMEM((128, HID), jnp.float32),
        pltpu.VMEM((128, HID), jnp.float32),
        pltpu.VMEM_SHARED((NACC, HID), jnp.float32),
        pltpu.SemaphoreType.DMA,
    ],
)


def _eps_body(ep_hbm, dstidx, whb_hbm, z_hbm, out_hbm,
              idxd_v, eb0, eb1, whb_v, acc, e0, e1, sc0, sc1):
    c = lax.axis_index("c")
    s = lax.axis_index("s")
    wid = s * 2 + c
    r0 = s * TROWS
    pltpu.sync_copy(z_hbm.at[pl.ds(r0, TROWS)], acc.at[pl.ds(r0, TROWS)])
    pltpu.sync_copy(dstidx.at[wid], idxd_v)
    pltpu.sync_copy(whb_hbm, whb_v)
    plsc.subcore_barrier()

    bufs = (eb0, eb1)
    esems = (e0, e1)
    ssems = (sc0, sc1)

    def ew(buf):
        def body(r, carry2):
            for c8 in range(8):
                sl = pl.ds(c8 * 16, 16)
                buf[r, sl] = buf[r, sl] + whb_v[sl]
            return carry2
        lax.fori_loop(0, 128, body, 0)

    def pair(p, carry):
        base = p * 2
        eds = [pltpu.async_copy(
            ep_hbm.at[pl.ds(wid * (NCHUNK * 128) + (base + b) * 128, 128)],
            bufs[b], esems[b]) for b in range(2)]
        sds = []
        for b in range(2):
            eds[b].wait()
            ew(bufs[b])
            sds.append(pltpu.async_copy(
                bufs[b], acc.at[idxd_v.at[base + b]], ssems[b], add=True))
        for b in range(2):
            sds[b].wait()
        return carry

    lax.fori_loop(0, NCHUNK // 2, pair, 0)
    plsc.subcore_barrier()
    pltpu.sync_copy(acc.at[pl.ds(r0, TROWS)], out_hbm.at[c, pl.ds(r0, TROWS)])


_eps_call = pl.kernel(
    _eps_body,
    out_type=jax.ShapeDtypeStruct((2, NACC, HID), jnp.float32),
    mesh=plsc.VectorSubcoreMesh(core_axis_name="c", subcore_axis_name="s"),
    scratch_types=[
        pltpu.VMEM((NCHUNK, 128), jnp.int32),
        pltpu.VMEM((128, HID), jnp.float32),
        pltpu.VMEM((128, HID), jnp.float32),
        pltpu.VMEM((HID,), jnp.float32),
        pltpu.VMEM_SHARED((NACC, HID), jnp.float32),
        pltpu.SemaphoreType.DMA,
        pltpu.SemaphoreType.DMA,
        pltpu.SemaphoreType.DMA,
        pltpu.SemaphoreType.DMA,
    ],
)


def _spmm_body(u_hbm, srcidx, dstidx, z_hbm, out_hbm, idxs_v, idxd_v,
               b0, b1, acc_sh, g0, g1, sc0, sc1):
    c = lax.axis_index("c")
    s = lax.axis_index("s")
    wid = s * 2 + c
    r0 = s * TROWS
    pltpu.sync_copy(z_hbm.at[pl.ds(r0, TROWS)], acc_sh.at[pl.ds(r0, TROWS)])
    pltpu.sync_copy(srcidx.at[wid], idxs_v)
    pltpu.sync_copy(dstidx.at[wid], idxd_v)
    plsc.subcore_barrier()

    bufs = (b0, b1)
    gsems = (g0, g1)
    ssems = (sc0, sc1)

    def group(q, carry):
        base = q * 2
        gds = [pltpu.async_copy(u_hbm.at[idxs_v.at[base + b]], bufs[b], gsems[b])
               for b in range(2)]
        sds = []
        for b in range(2):
            gds[b].wait()
            sds.append(pltpu.async_copy(
                bufs[b], acc_sh.at[idxd_v.at[base + b]], ssems[b], add=True))
        for b in range(2):
            sds[b].wait()
        return carry

    lax.fori_loop(0, NCHUNK // 2, group, 0)
    plsc.subcore_barrier()
    pltpu.sync_copy(acc_sh.at[pl.ds(r0, TROWS)], out_hbm.at[c, pl.ds(r0, TROWS)])


_spmm_call = pl.kernel(
    _spmm_body,
    out_type=jax.ShapeDtypeStruct((2, NACC, HID), jnp.float32),
    mesh=plsc.VectorSubcoreMesh(core_axis_name="c", subcore_axis_name="s"),
    scratch_types=[
        pltpu.VMEM((NCHUNK, 128), jnp.int32),
        pltpu.VMEM((NCHUNK, 128), jnp.int32),
        pltpu.VMEM((128, HID), jnp.float32),
        pltpu.VMEM((128, HID), jnp.float32),
        pltpu.VMEM_SHARED((NACC, HID), jnp.float32),
        pltpu.SemaphoreType.DMA,
        pltpu.SemaphoreType.DMA,
        pltpu.SemaphoreType.DMA,
        pltpu.SemaphoreType.DMA,
    ],
)


def _spmm(u, src_pad, dst_pad, zacc):
    t2 = _spmm_call(u, src_pad, dst_pad, zacc)
    return t2[0, :N], t2[1, :N]


def kernel(node_feats, edge_feats, edge_index, Win_w, Win_b, Wh_w, Wh_b,
           Wah_w, Wah_b, Wo_w, Wo_b):
    perm = jnp.argsort(edge_index[1])
    src = edge_index[0, perm]
    dst = edge_index[1, perm]
    edge_feats = edge_feats[perm]

    pad = EPAD - E
    src_pad = jnp.concatenate([src, jnp.zeros((pad,), jnp.int32)]).reshape(32, 40, 128)
    dst_pad = jnp.concatenate([dst, jnp.full((pad,), N, jnp.int32)]).reshape(32, 40, 128)

    win_x = Win_w[:, :DN]
    win_e = Win_w[:, DN:]

    node_proj = _mm_bias(node_feats, win_x, jnp.zeros((HID,), jnp.float32), ROW_BLK)
    p0 = _mm_bias(node_feats, Wah_w, Wah_b, ROW_BLK)

    ef_pad = jnp.concatenate(
        [edge_feats, jnp.zeros((pad, DE), jnp.float32)], axis=0)
    edge_proj_pad = _mm_bias(ef_pad, win_e, Win_b, EROW_BLK)

    zacc = jnp.zeros((NACC, HID), jnp.float32)
    s0o = _s0_call(node_proj, edge_proj_pad, src_pad, dst_pad, zacc)
    npo0, npo1 = _spmm(node_proj, src_pad, dst_pad, zacc)
    epo = _eps_call(edge_proj_pad, dst_pad, Wh_b, zacc)
    sbp = _sum4(npo0, npo1, epo[0, :N], epo[1, :N])
    u = _mm2(s0o[0, :N], s0o[1, :N], Wh_w)
    s = None
    for _ in range(NUM_MP):
        t0, t1 = _spmm(u, src_pad, dst_pad, zacc)
        s, u = _combine_mm(sbp, t0, t1, u, Wh_w)
    s6 = s

    h = _head(p0, s6, Wo_w, Wo_b)

    hpad = jnp.concatenate([h, jnp.zeros((NPAD - N, HID), jnp.float32)], axis=0)
    stats = _attn_stats(hpad)
    valid = (lax.broadcasted_iota(jnp.int32, (1, NPAD), 1) < N)
    m_f = jnp.where(valid, stats[0:8], 0.0)
    rz_f = jnp.where(valid, 1.0 / stats[8:16], 1.0)
    stats_f = jnp.concatenate([m_f, rz_f], axis=0)
    outp = _attn_out(hpad, stats_f)
    return outp[:N]


# speedup vs baseline: 1.0426x; 1.0426x over previous
import functools

import jax
import jax.numpy as jnp
from jax import lax
from jax.experimental import pallas as pl
from jax.experimental.pallas import tpu as pltpu
from jax.experimental.pallas import tpu_sc as plsc

N = 10000
E = 160000
DN = 128
DE = 16
HID = 128
NUM_MP = 6

NPAD = 10240
BI = 256
EPAD = 163840
ROW_BLK = 2000
EROW_BLK = 2048


def _mm_bias_body(x_ref, w_ref, b_ref, o_ref):
    o_ref[...] = lax.dot_general(
        x_ref[...], w_ref[...], (((1,), (1,)), ((), ()))) + b_ref[...]


def _mm_bias(x, w, b, row_blk):
    n, _ = x.shape
    dout = w.shape[0]
    grid = n // row_blk
    return pl.pallas_call(
        _mm_bias_body,
        grid=(grid,),
        in_specs=[
            pl.BlockSpec((row_blk, x.shape[1]), lambda i: (i, 0)),
            pl.BlockSpec(w.shape, lambda i: (0, 0)),
            pl.BlockSpec((1, dout), lambda i: (0, 0)),
        ],
        out_specs=pl.BlockSpec((row_blk, dout), lambda i: (i, 0)),
        out_shape=jax.ShapeDtypeStruct((n, dout), jnp.float32),
    )(x, w, b.reshape(1, dout))


def _combine_mm_body(sbp_ref, t0_ref, t1_ref, u_ref, w_ref, s_ref, up_ref):
    s = sbp_ref[...] + t0_ref[...] + t1_ref[...] - u_ref[...]
    s_ref[...] = s
    up_ref[...] = lax.dot_general(s, w_ref[...], (((1,), (1,)), ((), ())))


def _combine_mm(sbp, t0, t1, u, wh):
    grid = N // ROW_BLK
    blk = pl.BlockSpec((ROW_BLK, HID), lambda i: (i, 0))
    return pl.pallas_call(
        _combine_mm_body,
        grid=(grid,),
        in_specs=[blk, blk, blk, blk,
                  pl.BlockSpec((HID, HID), lambda i: (0, 0))],
        out_specs=[blk, blk],
        out_shape=[jax.ShapeDtypeStruct((N, HID), jnp.float32),
                   jax.ShapeDtypeStruct((N, HID), jnp.float32)],
    )(sbp, t0, t1, u, wh)


def _sum4_body(a_ref, b_ref, c_ref, d_ref, o_ref):
    o_ref[...] = a_ref[...] + b_ref[...] + c_ref[...] + d_ref[...]


def _sum4(a, b, c, d):
    grid = N // ROW_BLK
    blk = pl.BlockSpec((ROW_BLK, HID), lambda i: (i, 0))
    return pl.pallas_call(
        _sum4_body,
        grid=(grid,),
        in_specs=[blk, blk, blk, blk],
        out_specs=blk,
        out_shape=jax.ShapeDtypeStruct((N, HID), jnp.float32),
    )(a, b, c, d)


def _mm2_body(a_ref, b_ref, w_ref, o_ref):
    o_ref[...] = lax.dot_general(
        a_ref[...] + b_ref[...], w_ref[...], (((1,), (1,)), ((), ())))


def _mm2(a, b, w):
    grid = N // ROW_BLK
    blk = pl.BlockSpec((ROW_BLK, HID), lambda i: (i, 0))
    return pl.pallas_call(
        _mm2_body,
        grid=(grid,),
        in_specs=[blk, blk, pl.BlockSpec((HID, HID), lambda i: (0, 0))],
        out_specs=blk,
        out_shape=jax.ShapeDtypeStruct((N, HID), jnp.float32),
    )(a, b, w)


def _head_body(p0_ref, s6_ref, w_ref, b_ref, h_ref):
    pre = p0_ref[...] + s6_ref[...]
    h = lax.dot_general(pre, w_ref[...], (((1,), (1,)), ((), ()))) + b_ref[...]
    h_ref[...] = jnp.maximum(h, 0.0)


def _head(p0, s6, wo, wob):
    grid = N // ROW_BLK
    blk = pl.BlockSpec((ROW_BLK, HID), lambda i: (i, 0))
    return pl.pallas_call(
        _head_body,
        grid=(grid,),
        in_specs=[blk, blk, pl.BlockSpec((HID, HID), lambda i: (0, 0)),
                  pl.BlockSpec((1, HID), lambda i: (0, 0))],
        out_specs=blk,
        out_shape=jax.ShapeDtypeStruct((N, HID), jnp.float32),
    )(p0, s6, wo, wob.reshape(1, HID))


def _attn_stats_body(hb_ref, hf_ref, stats_ref):
    hb = hb_ref[...]
    hf = hf_ref[...]
    s = lax.dot_general(hb, hf, (((1,), (1,)), ((), ())))
    col = lax.broadcasted_iota(jnp.int32, s.shape, 1)
    s = jnp.where(col < N, s, -jnp.inf)
    m = jnp.max(s, axis=1)
    z = jnp.sum(jnp.exp(s - m[:, None]), axis=1)
    stats_ref[0:8, :] = jnp.broadcast_to(m[None, :], (8, BI))
    stats_ref[8:16, :] = jnp.broadcast_to(z[None, :], (8, BI))


def _attn_stats(hpad):
    grid = NPAD // BI
    return pl.pallas_call(
        _attn_stats_body,
        grid=(grid,),
        in_specs=[pl.BlockSpec((BI, HID), lambda i: (i, 0)),
                  pl.BlockSpec((NPAD, HID), lambda i: (0, 0))],
        out_specs=pl.BlockSpec((16, BI), lambda i: (0, i)),
        out_shape=jax.ShapeDtypeStruct((16, NPAD), jnp.float32),
    )(hpad, hpad)


def _attn_out_body(hb_ref, hf_ref, stats_ref, o_ref):
    hb = hb_ref[...]
    hf = hf_ref[...]
    s = lax.dot_general(hb, hf, (((1,), (1,)), ((), ())))
    m = stats_ref[0:1, :]
    rz = stats_ref[8:9, :]
    p = jnp.exp(s - m) * rz
    o_ref[...] = lax.dot_general(p, hf, (((1,), (0,)), ((), ()))) + hb


def _attn_out(hpad, stats):
    grid = NPAD // BI
    return pl.pallas_call(
        _attn_out_body,
        grid=(grid,),
        in_specs=[pl.BlockSpec((BI, HID), lambda i: (i, 0)),
                  pl.BlockSpec((NPAD, HID), lambda i: (0, 0)),
                  pl.BlockSpec((16, NPAD), lambda i: (0, 0))],
        out_specs=pl.BlockSpec((BI, HID), lambda i: (i, 0)),
        out_shape=jax.ShapeDtypeStruct((NPAD, HID), jnp.float32),
    )(hpad, hpad, stats)


NACC = 10112
TROWS = NACC // 16
NCHUNK = 40
EC = 80


def _s0_body(np_hbm, ep_hbm, srcidx, dstidx, z_hbm,
             out_hbm, idxs_v, idxd_v, grows_v, erows_v, acc, sem):
    c = lax.axis_index("c")
    s = lax.axis_index("s")
    wid = s * 2 + c
    r0 = s * TROWS
    pltpu.sync_copy(z_hbm.at[pl.ds(r0, TROWS)], acc.at[pl.ds(r0, TROWS)])
    pltpu.sync_copy(srcidx.at[wid], idxs_v)
    pltpu.sync_copy(dstidx.at[wid], idxd_v)
    plsc.subcore_barrier()

    def chunk(j, carry):
        gd = pltpu.async_copy(np_hbm.at[idxs_v.at[j]], grows_v, sem)
        pltpu.sync_copy(
            ep_hbm.at[pl.ds(wid * (NCHUNK * 128) + j * 128, 128)], erows_v)
        gd.wait()

        def ew(r, carry2):
            for c8 in range(8):
                sl = pl.ds(c8 * 16, 16)
                grows_v[r, sl] = jnp.maximum(grows_v[r, sl] + erows_v[r, sl], 0.0)
            return carry2

        lax.fori_loop(0, 128, ew, 0)
        pltpu.sync_copy(grows_v, acc.at[idxd_v.at[j]], add=True)
        return carry

    lax.fori_loop(0, NCHUNK, chunk, 0)
    plsc.subcore_barrier()
    pltpu.sync_copy(acc.at[pl.ds(r0, TROWS)], out_hbm.at[c, pl.ds(r0, TROWS)])


_s0_call = pl.kernel(
    _s0_body,
    out_type=jax.ShapeDtypeStruct((2, NACC, HID), jnp.float32),
    mesh=plsc.VectorSubcoreMesh(core_axis_name="c", subcore_axis_name="s"),
    scratch_types=[
        pltpu.VMEM((NCHUNK, 128), jnp.int32),
        pltpu.VMEM((NCHUNK, 128), jnp.int32),
        pltpu.VMEM((128, HID), jnp.float32),
        pltpu.VMEM((128, HID), jnp.float32),
        pltpu.VMEM_SHARED((NACC, HID), jnp.float32),
        pltpu.SemaphoreType.DMA,
    ],
)


def _eps_body(ep_hbm, dstidx, whb_hbm, z_hbm, out_hbm,
              idxd_v, eb0, eb1, whb_v, acc, e0, e1, sc0, sc1):
    c = lax.axis_index("c")
    s = lax.axis_index("s")
    wid = s * 2 + c
    r0 = s * TROWS
    pltpu.sync_copy(z_hbm.at[pl.ds(r0, TROWS)], acc.at[pl.ds(r0, TROWS)])
    pltpu.sync_copy(dstidx.at[wid], idxd_v)
    pltpu.sync_copy(whb_hbm, whb_v)
    plsc.subcore_barrier()

    bufs = (eb0, eb1)
    esems = (e0, e1)
    ssems = (sc0, sc1)

    def ew(buf):
        def body(r, carry2):
            for c8 in range(8):
                sl = pl.ds(c8 * 16, 16)
                buf[r, sl] = buf[r, sl] + whb_v[sl]
            return carry2
        lax.fori_loop(0, 128, body, 0)

    def pair(p, carry):
        base = p * 2
        eds = [pltpu.async_copy(
            ep_hbm.at[pl.ds(wid * (NCHUNK * 128) + (base + b) * 128, 128)],
            bufs[b], esems[b]) for b in range(2)]
        sds = []
        for b in range(2):
            eds[b].wait()
            ew(bufs[b])
            sds.append(pltpu.async_copy(
                bufs[b], acc.at[idxd_v.at[base + b]], ssems[b], add=True))
        for b in range(2):
            sds[b].wait()
        return carry

    lax.fori_loop(0, NCHUNK // 2, pair, 0)
    plsc.subcore_barrier()
    pltpu.sync_copy(acc.at[pl.ds(r0, TROWS)], out_hbm.at[c, pl.ds(r0, TROWS)])


_eps_call = pl.kernel(
    _eps_body,
    out_type=jax.ShapeDtypeStruct((2, NACC, HID), jnp.float32),
    mesh=plsc.VectorSubcoreMesh(core_axis_name="c", subcore_axis_name="s"),
    scratch_types=[
        pltpu.VMEM((NCHUNK, 128), jnp.int32),
        pltpu.VMEM((128, HID), jnp.float32),
        pltpu.VMEM((128, HID), jnp.float32),
        pltpu.VMEM((HID,), jnp.float32),
        pltpu.VMEM_SHARED((NACC, HID), jnp.float32),
        pltpu.SemaphoreType.DMA,
        pltpu.SemaphoreType.DMA,
        pltpu.SemaphoreType.DMA,
        pltpu.SemaphoreType.DMA,
    ],
)


def _spmm_body(u_hbm, srcidx, dstidx, z_hbm, out_hbm, idxs_v, idxd_v,
               buf_v, acc_sh, gsem):
    c = lax.axis_index("c")
    s = lax.axis_index("s")
    wid = s * 2 + c
    r0 = s * TROWS
    pltpu.sync_copy(z_hbm.at[pl.ds(r0, TROWS)], acc_sh.at[pl.ds(r0, TROWS)])
    pltpu.sync_copy(srcidx.at[wid], idxs_v)
    pltpu.sync_copy(dstidx.at[wid], idxd_v)
    plsc.subcore_barrier()

    def chunk(j, carry):
        pltpu.async_copy(u_hbm.at[idxs_v.at[j]], buf_v, gsem).wait()
        pltpu.sync_copy(buf_v, acc_sh.at[idxd_v.at[j]], add=True)
        return carry

    lax.fori_loop(0, NCHUNK, chunk, 0)
    plsc.subcore_barrier()
    pltpu.sync_copy(acc_sh.at[pl.ds(r0, TROWS)], out_hbm.at[c, pl.ds(r0, TROWS)])


_spmm_call = pl.kernel(
    _spmm_body,
    out_type=jax.ShapeDtypeStruct((2, NACC, HID), jnp.float32),
    mesh=plsc.VectorSubcoreMesh(core_axis_name="c", subcore_axis_name="s"),
    scratch_types=[
        pltpu.VMEM((NCHUNK, 128), jnp.int32),
        pltpu.VMEM((NCHUNK, 128), jnp.int32),
        pltpu.VMEM((128, HID), jnp.float32),
        pltpu.VMEM_SHARED((NACC, HID), jnp.float32),
        pltpu.SemaphoreType.DMA,
    ],
)


def _spmm(u, src_pad, dst_pad, zacc):
    t2 = _spmm_call(u, src_pad, dst_pad, zacc)
    return t2[0, :N], t2[1, :N]


def kernel(node_feats, edge_feats, edge_index, Win_w, Win_b, Wh_w, Wh_b,
           Wah_w, Wah_b, Wo_w, Wo_b):
    src = edge_index[0]
    dst = edge_index[1]

    pad = EPAD - E
    src_pad = jnp.concatenate([src, jnp.zeros((pad,), jnp.int32)]).reshape(32, 40, 128)
    dst_pad = jnp.concatenate([dst, jnp.full((pad,), N, jnp.int32)]).reshape(32, 40, 128)

    win_x = Win_w[:, :DN]
    win_e = Win_w[:, DN:]

    node_proj = _mm_bias(node_feats, win_x, jnp.zeros((HID,), jnp.float32), ROW_BLK)
    p0 = _mm_bias(node_feats, Wah_w, Wah_b, ROW_BLK)

    ef_pad = jnp.concatenate(
        [edge_feats, jnp.zeros((pad, DE), jnp.float32)], axis=0)
    edge_proj_pad = _mm_bias(ef_pad, win_e, Win_b, EROW_BLK)

    zacc = jnp.zeros((NACC, HID), jnp.float32)
    s0o = _s0_call(node_proj, edge_proj_pad, src_pad, dst_pad, zacc)
    npo0, npo1 = _spmm(node_proj, src_pad, dst_pad, zacc)
    epo = _eps_call(edge_proj_pad, dst_pad, Wh_b, zacc)
    sbp = _sum4(npo0, npo1, epo[0, :N], epo[1, :N])
    u = _mm2(s0o[0, :N], s0o[1, :N], Wh_w)
    s = None
    for _ in range(NUM_MP):
        t0, t1 = _spmm(u, src_pad, dst_pad, zacc)
        s, u = _combine_mm(sbp, t0, t1, u, Wh_w)
    s6 = s

    h = _head(p0, s6, Wo_w, Wo_b)

    hpad = jnp.concatenate([h, jnp.zeros((NPAD - N, HID), jnp.float32)], axis=0)
    stats = _attn_stats(hpad)
    valid = (lax.broadcasted_iota(jnp.int32, (1, NPAD), 1) < N)
    m_f = jnp.where(valid, stats[0:8], 0.0)
    rz_f = jnp.where(valid, 1.0 / stats[8:16], 1.0)
    stats_f = jnp.concatenate([m_f, rz_f], axis=0)
    outp = _attn_out(hpad, stats_f)
    return outp[:N]


# speedup vs baseline: 1.0534x; 1.0104x over previous
import functools

import jax
import jax.numpy as jnp
from jax import lax
from jax.experimental import pallas as pl
from jax.experimental.pallas import tpu as pltpu
from jax.experimental.pallas import tpu_sc as plsc

N = 10000
E = 160000
DN = 128
DE = 16
HID = 128
NUM_MP = 6

NPAD = 10240
BI = 256
EPAD = 163840
ROW_BLK = 2000
EROW_BLK = 2048


def _mm_bias_body(x_ref, w_ref, b_ref, o_ref):
    o_ref[...] = lax.dot_general(
        x_ref[...], w_ref[...], (((1,), (1,)), ((), ()))) + b_ref[...]


def _mm_bias(x, w, b, row_blk):
    n, _ = x.shape
    dout = w.shape[0]
    grid = n // row_blk
    return pl.pallas_call(
        _mm_bias_body,
        grid=(grid,),
        in_specs=[
            pl.BlockSpec((row_blk, x.shape[1]), lambda i: (i, 0)),
            pl.BlockSpec(w.shape, lambda i: (0, 0)),
            pl.BlockSpec((1, dout), lambda i: (0, 0)),
        ],
        out_specs=pl.BlockSpec((row_blk, dout), lambda i: (i, 0)),
        out_shape=jax.ShapeDtypeStruct((n, dout), jnp.float32),
    )(x, w, b.reshape(1, dout))


def _combine_mm_body(sbp_ref, t0_ref, t1_ref, u_ref, w_ref, s_ref, up_ref):
    s = sbp_ref[...] + t0_ref[...] + t1_ref[...] - u_ref[...]
    s_ref[...] = s
    up_ref[...] = lax.dot_general(s, w_ref[...], (((1,), (1,)), ((), ())))


def _combine_mm(sbp, t0, t1, u, wh):
    grid = N // ROW_BLK
    blk = pl.BlockSpec((ROW_BLK, HID), lambda i: (i, 0))
    return pl.pallas_call(
        _combine_mm_body,
        grid=(grid,),
        in_specs=[blk, blk, blk, blk,
                  pl.BlockSpec((HID, HID), lambda i: (0, 0))],
        out_specs=[blk, blk],
        out_shape=[jax.ShapeDtypeStruct((N, HID), jnp.float32),
                   jax.ShapeDtypeStruct((N, HID), jnp.float32)],
    )(sbp, t0, t1, u, wh)


def _sum2_body(a_ref, b_ref, o_ref):
    o_ref[...] = a_ref[...] + b_ref[...]


def _sum2(a, b):
    grid = N // ROW_BLK
    blk = pl.BlockSpec((ROW_BLK, HID), lambda i: (i, 0))
    return pl.pallas_call(
        _sum2_body,
        grid=(grid,),
        in_specs=[blk, blk],
        out_specs=blk,
        out_shape=jax.ShapeDtypeStruct((N, HID), jnp.float32),
    )(a, b)


def _mm2_body(a_ref, b_ref, w_ref, o_ref):
    o_ref[...] = lax.dot_general(
        a_ref[...] + b_ref[...], w_ref[...], (((1,), (1,)), ((), ())))


def _mm2(a, b, w):
    grid = N // ROW_BLK
    blk = pl.BlockSpec((ROW_BLK, HID), lambda i: (i, 0))
    return pl.pallas_call(
        _mm2_body,
        grid=(grid,),
        in_specs=[blk, blk, pl.BlockSpec((HID, HID), lambda i: (0, 0))],
        out_specs=blk,
        out_shape=jax.ShapeDtypeStruct((N, HID), jnp.float32),
    )(a, b, w)


def _head_body(p0_ref, s6_ref, w_ref, b_ref, h_ref):
    pre = p0_ref[...] + s6_ref[...]
    h = lax.dot_general(pre, w_ref[...], (((1,), (1,)), ((), ()))) + b_ref[...]
    h_ref[...] = jnp.maximum(h, 0.0)


def _head(p0, s6, wo, wob):
    grid = N // ROW_BLK
    blk = pl.BlockSpec((ROW_BLK, HID), lambda i: (i, 0))
    return pl.pallas_call(
        _head_body,
        grid=(grid,),
        in_specs=[blk, blk, pl.BlockSpec((HID, HID), lambda i: (0, 0)),
                  pl.BlockSpec((1, HID), lambda i: (0, 0))],
        out_specs=blk,
        out_shape=jax.ShapeDtypeStruct((N, HID), jnp.float32),
    )(p0, s6, wo, wob.reshape(1, HID))


def _attn_stats_body(hb_ref, hf_ref, stats_ref):
    hb = hb_ref[...]
    hf = hf_ref[...]
    s = lax.dot_general(hb, hf, (((1,), (1,)), ((), ())))
    col = lax.broadcasted_iota(jnp.int32, s.shape, 1)
    s = jnp.where(col < N, s, -jnp.inf)
    m = jnp.max(s, axis=1)
    z = jnp.sum(jnp.exp(s - m[:, None]), axis=1)
    stats_ref[0:8, :] = jnp.broadcast_to(m[None, :], (8, BI))
    stats_ref[8:16, :] = jnp.broadcast_to(z[None, :], (8, BI))


def _attn_stats(hpad):
    grid = NPAD // BI
    return pl.pallas_call(
        _attn_stats_body,
        grid=(grid,),
        in_specs=[pl.BlockSpec((BI, HID), lambda i: (i, 0)),
                  pl.BlockSpec((NPAD, HID), lambda i: (0, 0))],
        out_specs=pl.BlockSpec((16, BI), lambda i: (0, i)),
        out_shape=jax.ShapeDtypeStruct((16, NPAD), jnp.float32),
    )(hpad, hpad)


def _attn_out_body(hb_ref, hf_ref, stats_ref, o_ref):
    hb = hb_ref[...]
    hf = hf_ref[...]
    s = lax.dot_general(hb, hf, (((1,), (1,)), ((), ())))
    m = stats_ref[0:1, :]
    rz = stats_ref[8:9, :]
    p = jnp.exp(s - m) * rz
    o_ref[...] = lax.dot_general(p, hf, (((1,), (0,)), ((), ()))) + hb


def _attn_out(hpad, stats):
    grid = NPAD // BI
    return pl.pallas_call(
        _attn_out_body,
        grid=(grid,),
        in_specs=[pl.BlockSpec((BI, HID), lambda i: (i, 0)),
                  pl.BlockSpec((NPAD, HID), lambda i: (0, 0)),
                  pl.BlockSpec((16, NPAD), lambda i: (0, 0))],
        out_specs=pl.BlockSpec((BI, HID), lambda i: (i, 0)),
        out_shape=jax.ShapeDtypeStruct((NPAD, HID), jnp.float32),
    )(hpad, hpad, stats)


NACC = 10112
TROWS = NACC // 16
NCHUNK = 40
EC = 80


def _make_edge_body(relu_mode):
    def _edge_body(np_hbm, ep_hbm, srcidx, dstidx, whb_hbm, z_hbm,
                   out_hbm, idxs_v, idxd_v, grows_v, erows_v, whb_v, acc, sem):
        c = lax.axis_index("c")
        s = lax.axis_index("s")
        wid = s * 2 + c
        r0 = s * TROWS
        pltpu.sync_copy(z_hbm.at[pl.ds(r0, TROWS)], acc.at[pl.ds(r0, TROWS)])
        pltpu.sync_copy(srcidx.at[wid], idxs_v)
        pltpu.sync_copy(dstidx.at[wid], idxd_v)
        pltpu.sync_copy(whb_hbm, whb_v)
        plsc.subcore_barrier()

        def chunk(j, carry):
            gd = pltpu.async_copy(np_hbm.at[idxs_v.at[j]], grows_v, sem)
            pltpu.sync_copy(
                ep_hbm.at[pl.ds(wid * (NCHUNK * 128) + j * 128, 128)], erows_v)
            gd.wait()

            def ew(r, carry2):
                for c8 in range(8):
                    sl = pl.ds(c8 * 16, 16)
                    t = grows_v[r, sl] + erows_v[r, sl]
                    if relu_mode:
                        grows_v[r, sl] = jnp.maximum(t, 0.0)
                    else:
                        grows_v[r, sl] = t + whb_v[sl]
                return carry2

            lax.fori_loop(0, 128, ew, 0)
            pltpu.sync_copy(grows_v, acc.at[idxd_v.at[j]], add=True)
            return carry

        lax.fori_loop(0, NCHUNK, chunk, 0)
        plsc.subcore_barrier()
        pltpu.sync_copy(acc.at[pl.ds(r0, TROWS)], out_hbm.at[c, pl.ds(r0, TROWS)])

    return _edge_body


_edge_scratch = [
    pltpu.VMEM((NCHUNK, 128), jnp.int32),
    pltpu.VMEM((NCHUNK, 128), jnp.int32),
    pltpu.VMEM((128, HID), jnp.float32),
    pltpu.VMEM((128, HID), jnp.float32),
    pltpu.VMEM((HID,), jnp.float32),
    pltpu.VMEM_SHARED((NACC, HID), jnp.float32),
    pltpu.SemaphoreType.DMA,
]

_edge_call_relu = pl.kernel(
    _make_edge_body(True),
    out_type=jax.ShapeDtypeStruct((2, NACC, HID), jnp.float32),
    mesh=plsc.VectorSubcoreMesh(core_axis_name="c", subcore_axis_name="s"),
    scratch_types=_edge_scratch,
)

_edge_call_bias = pl.kernel(
    _make_edge_body(False),
    out_type=jax.ShapeDtypeStruct((2, NACC, HID), jnp.float32),
    mesh=plsc.VectorSubcoreMesh(core_axis_name="c", subcore_axis_name="s"),
    scratch_types=_edge_scratch,
)


def _spmm_body(u_hbm, srcidx, dstidx, z_hbm, out_hbm, idxs_v, idxd_v,
               buf_v, acc_sh, gsem):
    c = lax.axis_index("c")
    s = lax.axis_index("s")
    wid = s * 2 + c
    r0 = s * TROWS
    pltpu.sync_copy(z_hbm.at[pl.ds(r0, TROWS)], acc_sh.at[pl.ds(r0, TROWS)])
    pltpu.sync_copy(srcidx.at[wid], idxs_v)
    pltpu.sync_copy(dstidx.at[wid], idxd_v)
    plsc.subcore_barrier()

    def chunk(j, carry):
        pltpu.async_copy(u_hbm.at[idxs_v.at[j]], buf_v, gsem).wait()
        pltpu.sync_copy(buf_v, acc_sh.at[idxd_v.at[j]], add=True)
        return carry

    lax.fori_loop(0, NCHUNK, chunk, 0)
    plsc.subcore_barrier()
    pltpu.sync_copy(acc_sh.at[pl.ds(r0, TROWS)], out_hbm.at[c, pl.ds(r0, TROWS)])


_spmm_call = pl.kernel(
    _spmm_body,
    out_type=jax.ShapeDtypeStruct((2, NACC, HID), jnp.float32),
    mesh=plsc.VectorSubcoreMesh(core_axis_name="c", subcore_axis_name="s"),
    scratch_types=[
        pltpu.VMEM((NCHUNK, 128), jnp.int32),
        pltpu.VMEM((NCHUNK, 128), jnp.int32),
        pltpu.VMEM((128, HID), jnp.float32),
        pltpu.VMEM_SHARED((NACC, HID), jnp.float32),
        pltpu.SemaphoreType.DMA,
    ],
)


def _spmm(u, src_pad, dst_pad, zacc):
    t2 = _spmm_call(u, src_pad, dst_pad, zacc)
    return t2[0, :N], t2[1, :N]


def kernel(node_feats, edge_feats, edge_index, Win_w, Win_b, Wh_w, Wh_b,
           Wah_w, Wah_b, Wo_w, Wo_b):
    src = edge_index[0]
    dst = edge_index[1]

    pad = EPAD - E
    src_pad = jnp.concatenate([src, jnp.zeros((pad,), jnp.int32)]).reshape(32, 40, 128)
    dst_pad = jnp.concatenate([dst, jnp.full((pad,), N, jnp.int32)]).reshape(32, 40, 128)

    win_x = Win_w[:, :DN]
    win_e = Win_w[:, DN:]

    node_proj = _mm_bias(node_feats, win_x, jnp.zeros((HID,), jnp.float32), ROW_BLK)
    p0 = _mm_bias(node_feats, Wah_w, Wah_b, ROW_BLK)

    ef_pad = jnp.concatenate(
        [edge_feats, jnp.zeros((pad, DE), jnp.float32)], axis=0)
    edge_proj_pad = _mm_bias(ef_pad, win_e, Win_b, EROW_BLK)

    zacc = jnp.zeros((NACC, HID), jnp.float32)
    s0o = _edge_call_relu(node_proj, edge_proj_pad, src_pad, dst_pad, Wh_b, zacc)
    sbo = _edge_call_bias(node_proj, edge_proj_pad, src_pad, dst_pad, Wh_b, zacc)
    sbp = _sum2(sbo[0, :N], sbo[1, :N])
    u = _mm2(s0o[0, :N], s0o[1, :N], Wh_w)
    s = None
    for _ in range(NUM_MP):
        t0, t1 = _spmm(u, src_pad, dst_pad, zacc)
        s, u = _combine_mm(sbp, t0, t1, u, Wh_w)
    s6 = s

    h = _head(p0, s6, Wo_w, Wo_b)

    hpad = jnp.concatenate([h, jnp.zeros((NPAD - N, HID), jnp.float32)], axis=0)
    stats = _attn_stats(hpad)
    valid = (lax.broadcasted_iota(jnp.int32, (1, NPAD), 1) < N)
    m_f = jnp.where(valid, stats[0:8], 0.0)
    rz_f = jnp.where(valid, 1.0 / stats[8:16], 1.0)
    stats_f = jnp.concatenate([m_f, rz_f], axis=0)
    outp = _attn_out(hpad, stats_f)
    return outp[:N]


# speedup vs baseline: 1.1110x; 1.0547x over previous
import functools

import jax
import jax.numpy as jnp
from jax import lax
from jax.experimental import pallas as pl
from jax.experimental.pallas import tpu as pltpu
from jax.experimental.pallas import tpu_sc as plsc

N = 10000
E = 160000
DN = 128
DE = 16
HID = 128
NUM_MP = 6

NPAD = 10240
BI = 256
EPAD = 163840
ROW_BLK = 2000
EROW_BLK = 2048


def _mm_bias_body(x_ref, w_ref, b_ref, o_ref):
    o_ref[...] = lax.dot_general(
        x_ref[...], w_ref[...], (((1,), (1,)), ((), ()))) + b_ref[...]


def _mm_bias(x, w, b, row_blk):
    n, _ = x.shape
    dout = w.shape[0]
    grid = n // row_blk
    return pl.pallas_call(
        _mm_bias_body,
        grid=(grid,),
        in_specs=[
            pl.BlockSpec((row_blk, x.shape[1]), lambda i: (i, 0)),
            pl.BlockSpec(w.shape, lambda i: (0, 0)),
            pl.BlockSpec((1, dout), lambda i: (0, 0)),
        ],
        out_specs=pl.BlockSpec((row_blk, dout), lambda i: (i, 0)),
        out_shape=jax.ShapeDtypeStruct((n, dout), jnp.float32),
    )(x, w, b.reshape(1, dout))


def _combine_mm_body(sba_ref, sbb_ref, t0_ref, t1_ref, u_ref, w_ref,
                     s_ref, up_ref):
    s = (sba_ref[...] + sbb_ref[...] + t0_ref[...] + t1_ref[...]
         - u_ref[...])
    s_ref[...] = s
    up_ref[...] = lax.dot_general(s, w_ref[...], (((1,), (1,)), ((), ())))


def _combine_mm(sba, sbb, t0, t1, u, wh):
    grid = N // ROW_BLK
    blk = pl.BlockSpec((ROW_BLK, HID), lambda i: (i, 0))
    return pl.pallas_call(
        _combine_mm_body,
        grid=(grid,),
        in_specs=[blk, blk, blk, blk, blk,
                  pl.BlockSpec((HID, HID), lambda i: (0, 0))],
        out_specs=[blk, blk],
        out_shape=[jax.ShapeDtypeStruct((N, HID), jnp.float32),
                   jax.ShapeDtypeStruct((N, HID), jnp.float32)],
    )(sba, sbb, t0, t1, u, wh)


def _sum2_body(a_ref, b_ref, o_ref):
    o_ref[...] = a_ref[...] + b_ref[...]


def _sum2(a, b):
    grid = N // ROW_BLK
    blk = pl.BlockSpec((ROW_BLK, HID), lambda i: (i, 0))
    return pl.pallas_call(
        _sum2_body,
        grid=(grid,),
        in_specs=[blk, blk],
        out_specs=blk,
        out_shape=jax.ShapeDtypeStruct((N, HID), jnp.float32),
    )(a, b)


def _mm2_body(a_ref, b_ref, w_ref, o_ref):
    o_ref[...] = lax.dot_general(
        a_ref[...] + b_ref[...], w_ref[...], (((1,), (1,)), ((), ())))


def _mm2(a, b, w):
    grid = N // ROW_BLK
    blk = pl.BlockSpec((ROW_BLK, HID), lambda i: (i, 0))
    return pl.pallas_call(
        _mm2_body,
        grid=(grid,),
        in_specs=[blk, blk, pl.BlockSpec((HID, HID), lambda i: (0, 0))],
        out_specs=blk,
        out_shape=jax.ShapeDtypeStruct((N, HID), jnp.float32),
    )(a, b, w)


def _head_body(p0_ref, s6_ref, w_ref, b_ref, h_ref):
    pre = p0_ref[...] + s6_ref[...]
    h = lax.dot_general(pre, w_ref[...], (((1,), (1,)), ((), ()))) + b_ref[...]
    h_ref[...] = jnp.maximum(h, 0.0)


def _head(p0, s6, wo, wob):
    grid = N // ROW_BLK
    blk = pl.BlockSpec((ROW_BLK, HID), lambda i: (i, 0))
    return pl.pallas_call(
        _head_body,
        grid=(grid,),
        in_specs=[blk, blk, pl.BlockSpec((HID, HID), lambda i: (0, 0)),
                  pl.BlockSpec((1, HID), lambda i: (0, 0))],
        out_specs=blk,
        out_shape=jax.ShapeDtypeStruct((N, HID), jnp.float32),
    )(p0, s6, wo, wob.reshape(1, HID))


def _attn_stats_body(hb_ref, hf_ref, stats_ref):
    hb = hb_ref[...]
    hf = hf_ref[...]
    s = lax.dot_general(hb, hf, (((1,), (1,)), ((), ())))
    col = lax.broadcasted_iota(jnp.int32, s.shape, 1)
    s = jnp.where(col < N, s, -jnp.inf)
    m = jnp.max(s, axis=1)
    z = jnp.sum(jnp.exp(s - m[:, None]), axis=1)
    stats_ref[0:8, :] = jnp.broadcast_to(m[None, :], (8, BI))
    stats_ref[8:16, :] = jnp.broadcast_to(z[None, :], (8, BI))


def _attn_stats(hpad):
    grid = NPAD // BI
    return pl.pallas_call(
        _attn_stats_body,
        grid=(grid,),
        in_specs=[pl.BlockSpec((BI, HID), lambda i: (i, 0)),
                  pl.BlockSpec((NPAD, HID), lambda i: (0, 0))],
        out_specs=pl.BlockSpec((16, BI), lambda i: (0, i)),
        out_shape=jax.ShapeDtypeStruct((16, NPAD), jnp.float32),
    )(hpad, hpad)


def _attn_out_body(hb_ref, hf_ref, stats_ref, o_ref):
    hb = hb_ref[...]
    hf = hf_ref[...]
    s = lax.dot_general(hb, hf, (((1,), (1,)), ((), ())))
    m = stats_ref[0:1, :]
    rz = stats_ref[8:9, :]
    p = jnp.exp(s - m) * rz
    o_ref[...] = lax.dot_general(p, hf, (((1,), (0,)), ((), ()))) + hb


def _attn_out(hpad, stats):
    grid = NPAD // BI
    return pl.pallas_call(
        _attn_out_body,
        grid=(grid,),
        in_specs=[pl.BlockSpec((BI, HID), lambda i: (i, 0)),
                  pl.BlockSpec((NPAD, HID), lambda i: (0, 0)),
                  pl.BlockSpec((16, NPAD), lambda i: (0, 0))],
        out_specs=pl.BlockSpec((BI, HID), lambda i: (i, 0)),
        out_shape=jax.ShapeDtypeStruct((NPAD, HID), jnp.float32),
    )(hpad, hpad, stats)


NACC = 10112
TROWS = NACC // 16
NCHUNK = 40
EC = 80


def _make_edge_body(relu_mode):
    def _edge_body(np_hbm, ep_hbm, srcidx, dstidx, whb_hbm, z_hbm,
                   out_hbm, idxs_v, idxd_v, grows_v, erows_v, whb_v, acc, sem):
        c = lax.axis_index("c")
        s = lax.axis_index("s")
        wid = s * 2 + c
        r0 = s * TROWS
        pltpu.sync_copy(z_hbm.at[pl.ds(r0, TROWS)], acc.at[pl.ds(r0, TROWS)])
        pltpu.sync_copy(srcidx.at[wid], idxs_v)
        pltpu.sync_copy(dstidx.at[wid], idxd_v)
        pltpu.sync_copy(whb_hbm, whb_v)
        plsc.subcore_barrier()

        def chunk(j, carry):
            gd = pltpu.async_copy(np_hbm.at[idxs_v.at[j]], grows_v, sem)
            pltpu.sync_copy(
                ep_hbm.at[pl.ds(wid * (NCHUNK * 128) + j * 128, 128)], erows_v)
            gd.wait()

            def ew(r, carry2):
                for c8 in range(8):
                    sl = pl.ds(c8 * 16, 16)
                    t = grows_v[r, sl] + erows_v[r, sl]
                    if relu_mode:
                        grows_v[r, sl] = jnp.maximum(t, 0.0)
                    else:
                        grows_v[r, sl] = t + whb_v[sl]
                return carry2

            lax.fori_loop(0, 128, ew, 0)
            pltpu.sync_copy(grows_v, acc.at[idxd_v.at[j]], add=True)
            return carry

        lax.fori_loop(0, NCHUNK, chunk, 0)
        plsc.subcore_barrier()
        pltpu.sync_copy(acc.at[pl.ds(r0, TROWS)], out_hbm.at[c, pl.ds(r0, TROWS)])

    return _edge_body


_edge_scratch = [
    pltpu.VMEM((NCHUNK, 128), jnp.int32),
    pltpu.VMEM((NCHUNK, 128), jnp.int32),
    pltpu.VMEM((128, HID), jnp.float32),
    pltpu.VMEM((128, HID), jnp.float32),
    pltpu.VMEM((HID,), jnp.float32),
    pltpu.VMEM_SHARED((NACC, HID), jnp.float32),
    pltpu.SemaphoreType.DMA,
]

_edge_call_relu = pl.kernel(
    _make_edge_body(True),
    out_type=jax.ShapeDtypeStruct((2, NACC, HID), jnp.float32),
    mesh=plsc.VectorSubcoreMesh(core_axis_name="c", subcore_axis_name="s"),
    scratch_types=_edge_scratch,
)

_edge_call_bias = pl.kernel(
    _make_edge_body(False),
    out_type=jax.ShapeDtypeStruct((2, NACC, HID), jnp.float32),
    mesh=plsc.VectorSubcoreMesh(core_axis_name="c", subcore_axis_name="s"),
    scratch_types=_edge_scratch,
)


def _spmm_body(u_hbm, srcidx, dstidx, z_hbm, out_hbm, idxs_v, idxd_v,
               buf_v, acc_sh, gsem):
    c = lax.axis_index("c")
    s = lax.axis_index("s")
    wid = s * 2 + c
    r0 = s * TROWS
    pltpu.sync_copy(z_hbm.at[pl.ds(r0, TROWS)], acc_sh.at[pl.ds(r0, TROWS)])
    pltpu.sync_copy(srcidx.at[wid], idxs_v)
    pltpu.sync_copy(dstidx.at[wid], idxd_v)
    plsc.subcore_barrier()

    def chunk(j, carry):
        pltpu.async_copy(u_hbm.at[idxs_v.at[j]], buf_v, gsem).wait()
        pltpu.sync_copy(buf_v, acc_sh.at[idxd_v.at[j]], add=True)
        return carry

    lax.fori_loop(0, NCHUNK, chunk, 0)
    plsc.subcore_barrier()
    pltpu.sync_copy(acc_sh.at[pl.ds(r0, TROWS)], out_hbm.at[c, pl.ds(r0, TROWS)])


_spmm_call = pl.kernel(
    _spmm_body,
    out_type=jax.ShapeDtypeStruct((2, NACC, HID), jnp.float32),
    mesh=plsc.VectorSubcoreMesh(core_axis_name="c", subcore_axis_name="s"),
    scratch_types=[
        pltpu.VMEM((NCHUNK, 128), jnp.int32),
        pltpu.VMEM((NCHUNK, 128), jnp.int32),
        pltpu.VMEM((128, HID), jnp.float32),
        pltpu.VMEM_SHARED((NACC, HID), jnp.float32),
        pltpu.SemaphoreType.DMA,
    ],
)


def _spmm(u, src_pad, dst_pad, zacc):
    t2 = _spmm_call(u, src_pad, dst_pad, zacc)
    return t2[0, :N], t2[1, :N]


def kernel(node_feats, edge_feats, edge_index, Win_w, Win_b, Wh_w, Wh_b,
           Wah_w, Wah_b, Wo_w, Wo_b):
    src = edge_index[0]
    dst = edge_index[1]

    pad = EPAD - E
    src_pad = jnp.concatenate([src, jnp.zeros((pad,), jnp.int32)]).reshape(32, 40, 128)
    dst_pad = jnp.concatenate([dst, jnp.full((pad,), N, jnp.int32)]).reshape(32, 40, 128)

    win_x = Win_w[:, :DN]
    win_e = Win_w[:, DN:]

    node_proj = _mm_bias(node_feats, win_x, jnp.zeros((HID,), jnp.float32), ROW_BLK)
    p0 = _mm_bias(node_feats, Wah_w, Wah_b, ROW_BLK)

    ef_pad = jnp.concatenate(
        [edge_feats, jnp.zeros((pad, DE), jnp.float32)], axis=0)
    edge_proj_pad = _mm_bias(ef_pad, win_e, Win_b, EROW_BLK)

    zacc = jnp.zeros((NACC, HID), jnp.float32)
    s0o = _edge_call_relu(node_proj, edge_proj_pad, src_pad, dst_pad, Wh_b, zacc)
    sbo = _edge_call_bias(node_proj, edge_proj_pad, src_pad, dst_pad, Wh_b, zacc)
    sba, sbb = sbo[0, :N], sbo[1, :N]
    u = _mm2(s0o[0, :N], s0o[1, :N], Wh_w)
    s = None
    for _ in range(NUM_MP):
        t0, t1 = _spmm(u, src_pad, dst_pad, zacc)
        s, u = _combine_mm(sba, sbb, t0, t1, u, Wh_w)
    s6 = s

    h = _head(p0, s6, Wo_w, Wo_b)

    hpad = jnp.concatenate([h, jnp.zeros((NPAD - N, HID), jnp.float32)], axis=0)
    stats = _attn_stats(hpad)
    valid = (lax.broadcasted_iota(jnp.int32, (1, NPAD), 1) < N)
    m_f = jnp.where(valid, stats[0:8], 0.0)
    rz_f = jnp.where(valid, 1.0 / stats[8:16], 1.0)
    stats_f = jnp.concatenate([m_f, rz_f], axis=0)
    outp = _attn_out(hpad, stats_f)
    return outp[:N]


# speedup vs baseline: 1.1273x; 1.0146x over previous
import functools

import jax
import jax.numpy as jnp
from jax import lax
from jax.experimental import pallas as pl
from jax.experimental.pallas import tpu as pltpu
from jax.experimental.pallas import tpu_sc as plsc

N = 10000
E = 160000
DN = 128
DE = 16
HID = 128
NUM_MP = 6

NPAD = 10240
BI = 256
EPAD = 163840
ROW_BLK = 2000
EROW_BLK = 2048


def _mm_bias_body(x_ref, w_ref, b_ref, o_ref):
    o_ref[...] = lax.dot_general(
        x_ref[...], w_ref[...], (((1,), (1,)), ((), ()))) + b_ref[...]


def _mm_bias(x, w, b, row_blk):
    n, _ = x.shape
    dout = w.shape[0]
    grid = n // row_blk
    return pl.pallas_call(
        _mm_bias_body,
        grid=(grid,),
        in_specs=[
            pl.BlockSpec((row_blk, x.shape[1]), lambda i: (i, 0)),
            pl.BlockSpec(w.shape, lambda i: (0, 0)),
            pl.BlockSpec((1, dout), lambda i: (0, 0)),
        ],
        out_specs=pl.BlockSpec((row_blk, dout), lambda i: (i, 0)),
        out_shape=jax.ShapeDtypeStruct((n, dout), jnp.float32),
    )(x, w, b.reshape(1, dout))


def _combine_mm_body(sba_ref, sbb_ref, t0_ref, t1_ref, u_ref, w_ref,
                     s_ref, up_ref):
    s = (sba_ref[...] + sbb_ref[...] + t0_ref[...] + t1_ref[...]
         - u_ref[...])
    s_ref[...] = s
    up_ref[...] = lax.dot_general(s, w_ref[...], (((1,), (1,)), ((), ())))


def _combine_mm(sba, sbb, t0, t1, u, wh):
    grid = N // ROW_BLK
    blk = pl.BlockSpec((ROW_BLK, HID), lambda i: (i, 0))
    return pl.pallas_call(
        _combine_mm_body,
        grid=(grid,),
        in_specs=[blk, blk, blk, blk, blk,
                  pl.BlockSpec((HID, HID), lambda i: (0, 0))],
        out_specs=[blk, blk],
        out_shape=[jax.ShapeDtypeStruct((N, HID), jnp.float32),
                   jax.ShapeDtypeStruct((N, HID), jnp.float32)],
    )(sba, sbb, t0, t1, u, wh)


def _mm2_body(a_ref, b_ref, w_ref, o_ref):
    o_ref[...] = lax.dot_general(
        a_ref[...] + b_ref[...], w_ref[...], (((1,), (1,)), ((), ())))


def _mm2(a, b, w):
    grid = N // ROW_BLK
    blk = pl.BlockSpec((ROW_BLK, HID), lambda i: (i, 0))
    return pl.pallas_call(
        _mm2_body,
        grid=(grid,),
        in_specs=[blk, blk, pl.BlockSpec((HID, HID), lambda i: (0, 0))],
        out_specs=blk,
        out_shape=jax.ShapeDtypeStruct((N, HID), jnp.float32),
    )(a, b, w)


def _head_body(p0_ref, s6_ref, w_ref, b_ref, h_ref):
    pre = p0_ref[...] + s6_ref[...]
    h = lax.dot_general(pre, w_ref[...], (((1,), (1,)), ((), ()))) + b_ref[...]
    h_ref[...] = jnp.maximum(h, 0.0)


def _head(p0, s6, wo, wob):
    grid = N // ROW_BLK
    blk = pl.BlockSpec((ROW_BLK, HID), lambda i: (i, 0))
    return pl.pallas_call(
        _head_body,
        grid=(grid,),
        in_specs=[blk, blk, pl.BlockSpec((HID, HID), lambda i: (0, 0)),
                  pl.BlockSpec((1, HID), lambda i: (0, 0))],
        out_specs=blk,
        out_shape=jax.ShapeDtypeStruct((N, HID), jnp.float32),
    )(p0, s6, wo, wob.reshape(1, HID))


def _attn_stats_body(hb_ref, hf_ref, stats_ref):
    hb = hb_ref[...]
    hf = hf_ref[...]
    s = lax.dot_general(hb, hf, (((1,), (1,)), ((), ())))
    col = lax.broadcasted_iota(jnp.int32, s.shape, 1)
    s = jnp.where(col < N, s, -jnp.inf)
    m = jnp.max(s, axis=1)
    z = jnp.sum(jnp.exp(s - m[:, None]), axis=1)
    stats_ref[0:8, :] = jnp.broadcast_to(m[None, :], (8, BI))
    stats_ref[8:16, :] = jnp.broadcast_to(z[None, :], (8, BI))


def _attn_stats(hpad):
    grid = NPAD // BI
    return pl.pallas_call(
        _attn_stats_body,
        grid=(grid,),
        in_specs=[pl.BlockSpec((BI, HID), lambda i: (i, 0)),
                  pl.BlockSpec((NPAD, HID), lambda i: (0, 0))],
        out_specs=pl.BlockSpec((16, BI), lambda i: (0, i)),
        out_shape=jax.ShapeDtypeStruct((16, NPAD), jnp.float32),
    )(hpad, hpad)


def _attn_out_body(hb_ref, hf_ref, stats_ref, o_ref):
    hb = hb_ref[...]
    hf = hf_ref[...]
    s = lax.dot_general(hb, hf, (((1,), (1,)), ((), ())))
    m = stats_ref[0:1, :]
    rz = stats_ref[8:9, :]
    p = jnp.exp(s - m) * rz
    o_ref[...] = lax.dot_general(p, hf, (((1,), (0,)), ((), ()))) + hb


def _attn_out(hpad, stats):
    grid = NPAD // BI
    return pl.pallas_call(
        _attn_out_body,
        grid=(grid,),
        in_specs=[pl.BlockSpec((BI, HID), lambda i: (i, 0)),
                  pl.BlockSpec((NPAD, HID), lambda i: (0, 0)),
                  pl.BlockSpec((16, NPAD), lambda i: (0, 0))],
        out_specs=pl.BlockSpec((BI, HID), lambda i: (i, 0)),
        out_shape=jax.ShapeDtypeStruct((NPAD, HID), jnp.float32),
    )(hpad, hpad, stats)


NACC = 10112
TROWS = NACC // 16
NCHUNK = 40
EC = 80


def _make_edge_body(relu_mode):
    def _edge_body(np_hbm, ep_hbm, srcidx, dstidx, whb_hbm, z_hbm,
                   out_hbm, idxs_v, idxd_v, grows_v, erows_v, whb_v, acc, sem):
        c = lax.axis_index("c")
        s = lax.axis_index("s")
        wid = s * 2 + c
        r0 = s * TROWS
        pltpu.sync_copy(z_hbm.at[pl.ds(r0, TROWS)], acc.at[pl.ds(r0, TROWS)])
        pltpu.sync_copy(srcidx.at[wid], idxs_v)
        pltpu.sync_copy(dstidx.at[wid], idxd_v)
        pltpu.sync_copy(whb_hbm, whb_v)
        plsc.subcore_barrier()

        def chunk(j, carry):
            gd = pltpu.async_copy(np_hbm.at[idxs_v.at[j]], grows_v, sem)
            pltpu.sync_copy(
                ep_hbm.at[pl.ds(wid * (NCHUNK * 128) + j * 128, 128)], erows_v)
            gd.wait()

            def ew(r, carry2):
                for c8 in range(8):
                    sl = pl.ds(c8 * 16, 16)
                    t = grows_v[r, sl] + erows_v[r, sl]
                    if relu_mode:
                        grows_v[r, sl] = jnp.maximum(t, 0.0)
                    else:
                        grows_v[r, sl] = t + whb_v[sl]
                return carry2

            lax.fori_loop(0, 128, ew, 0)
            pltpu.sync_copy(grows_v, acc.at[idxd_v.at[j]], add=True)
            return carry

        lax.fori_loop(0, NCHUNK, chunk, 0)
        plsc.subcore_barrier()
        pltpu.sync_copy(acc.at[pl.ds(r0, TROWS)], out_hbm.at[c, pl.ds(r0, TROWS)])

    return _edge_body


_edge_scratch = [
    pltpu.VMEM((NCHUNK, 128), jnp.int32),
    pltpu.VMEM((NCHUNK, 128), jnp.int32),
    pltpu.VMEM((128, HID), jnp.float32),
    pltpu.VMEM((128, HID), jnp.float32),
    pltpu.VMEM((HID,), jnp.float32),
    pltpu.VMEM_SHARED((NACC, HID), jnp.float32),
    pltpu.SemaphoreType.DMA,
]

_edge_call_relu = pl.kernel(
    _make_edge_body(True),
    out_type=jax.ShapeDtypeStruct((2, NACC, HID), jnp.float32),
    mesh=plsc.VectorSubcoreMesh(core_axis_name="c", subcore_axis_name="s"),
    scratch_types=_edge_scratch,
)

_edge_call_bias = pl.kernel(
    _make_edge_body(False),
    out_type=jax.ShapeDtypeStruct((2, NACC, HID), jnp.float32),
    mesh=plsc.VectorSubcoreMesh(core_axis_name="c", subcore_axis_name="s"),
    scratch_types=_edge_scratch,
)


def _spmm_body(u_hbm, srcidx, dstidx, z_hbm, out_hbm, idxs_v, idxd_v,
               buf_v, acc_sh, gsem):
    c = lax.axis_index("c")
    s = lax.axis_index("s")
    wid = s * 2 + c
    r0 = s * TROWS
    pltpu.sync_copy(z_hbm.at[pl.ds(r0, TROWS)], acc_sh.at[pl.ds(r0, TROWS)])
    pltpu.sync_copy(srcidx.at[wid], idxs_v)
    pltpu.sync_copy(dstidx.at[wid], idxd_v)
    plsc.subcore_barrier()

    def chunk(j, carry):
        pltpu.async_copy(u_hbm.at[idxs_v.at[j]], buf_v, gsem).wait()
        pltpu.sync_copy(buf_v, acc_sh.at[idxd_v.at[j]], add=True)
        return carry

    lax.fori_loop(0, NCHUNK, chunk, 0)
    plsc.subcore_barrier()
    pltpu.sync_copy(acc_sh.at[pl.ds(r0, TROWS)], out_hbm.at[c, pl.ds(r0, TROWS)])


_spmm_call = pl.kernel(
    _spmm_body,
    out_type=jax.ShapeDtypeStruct((2, NACC, HID), jnp.float32),
    mesh=plsc.VectorSubcoreMesh(core_axis_name="c", subcore_axis_name="s"),
    scratch_types=[
        pltpu.VMEM((NCHUNK, 128), jnp.int32),
        pltpu.VMEM((NCHUNK, 128), jnp.int32),
        pltpu.VMEM((128, HID), jnp.float32),
        pltpu.VMEM_SHARED((NACC, HID), jnp.float32),
        pltpu.SemaphoreType.DMA,
    ],
)


def _spmm(u, src_pad, dst_pad, zacc):
    t2 = _spmm_call(u, src_pad, dst_pad, zacc)
    return t2[0, :N], t2[1, :N]


def kernel(node_feats, edge_feats, edge_index, Win_w, Win_b, Wh_w, Wh_b,
           Wah_w, Wah_b, Wo_w, Wo_b):
    src = edge_index[0]
    dst = edge_index[1]

    pad = EPAD - E
    src_pad = jnp.concatenate([src, jnp.zeros((pad,), jnp.int32)]).reshape(32, 40, 128)
    dst_pad = jnp.concatenate([dst, jnp.full((pad,), N, jnp.int32)]).reshape(32, 40, 128)

    win_x = Win_w[:, :DN]
    win_e = Win_w[:, DN:]

    node_proj = _mm_bias(node_feats, win_x, jnp.zeros((HID,), jnp.float32), ROW_BLK)
    p0 = _mm_bias(node_feats, Wah_w, Wah_b, ROW_BLK)

    ef_pad = jnp.concatenate(
        [edge_feats, jnp.zeros((pad, DE), jnp.float32)], axis=0)
    edge_proj_pad = _mm_bias(ef_pad, win_e, Win_b, EROW_BLK)

    zacc = jnp.zeros((NACC, HID), jnp.float32)
    s0o = _edge_call_relu(node_proj, edge_proj_pad, src_pad, dst_pad, Wh_b, zacc)
    sbo = _edge_call_bias(node_proj, edge_proj_pad, src_pad, dst_pad, Wh_b, zacc)
    sba, sbb = sbo[0, :N], sbo[1, :N]
    u = _mm2(s0o[0, :N], s0o[1, :N], Wh_w)
    s = None
    for _ in range(NUM_MP):
        t0, t1 = _spmm(u, src_pad, dst_pad, zacc)
        s, u = _combine_mm(sba, sbb, t0, t1, u, Wh_w)
    s6 = s

    h = _head(p0, s6, Wo_w, Wo_b)

    hpad = jnp.concatenate([h, jnp.zeros((NPAD - N, HID), jnp.float32)], axis=0)
    stats = _attn_stats(hpad)
    valid = (lax.broadcasted_iota(jnp.int32, (1, NPAD), 1) < N)
    m_f = jnp.where(valid, stats[0:8], 0.0)
    rz_f = jnp.where(valid, 1.0 / stats[8:16], 1.0)
    stats_f = jnp.concatenate([m_f, rz_f], axis=0)
    outp = _attn_out(hpad, stats_f)
    return outp[:N]
